# baseline (speedup 1.0000x reference)
import dataclasses
import functools

import jax
import jax.numpy as jnp
from jax import lax
from jax.experimental import pallas as pl
from jax.experimental.pallas import tpu as pltpu
from jax.experimental.pallas import tpu_sc as plsc

B, T, N, D = 8, 24, 1024, 3
DM = 256
K = 16
L = 4
BN = B * N
E = BN * K
TD = T * D
VD = (T - 1) * D

_NW = 32
_EPW = E // _NW
_GCH = 128
_RPW = BN // _NW

_mesh = plsc.VectorSubcoreMesh(core_axis_name="c", subcore_axis_name="s")

_sc_params = pltpu.CompilerParams()
if "needs_layout_passes" in pltpu.CompilerParams.__dataclass_fields__:
    _sc_params = dataclasses.replace(_sc_params, needs_layout_passes=False)


def _prep_body(pos_ref, mp_ref, nw0_ref, nb0_ref, nw1_ref, nb1_ref, ewc_ref,
               wa_ref, wb_ref, eb0_ref,
               h_ref, c_ref, s_ref, t_ref, delta_ref):
    x = pos_ref[...]
    h1 = jnp.maximum(
        jnp.dot(x, nw0_ref[...], preferred_element_type=jnp.float32)
        + nb0_ref[...], 0.0)
    h = (jnp.dot(h1, nw1_ref[...], preferred_element_type=jnp.float32)
         + nb1_ref[...])
    h_ref[...] = h
    c = jnp.dot(mp_ref[...], ewc_ref[...], preferred_element_type=jnp.float32)
    c_ref[...] = c
    s_ref[...] = (jnp.dot(h, wa_ref[...], preferred_element_type=jnp.float32)
                  + c + eb0_ref[...])
    t_ref[...] = (jnp.dot(h, wb_ref[...], preferred_element_type=jnp.float32)
                  - c).astype(jnp.bfloat16)
    delta_ref[...] = x[:, D:] - x[:, :TD - D]


def _prep(pos_t, mp_bn, nw0, nb0, nw1, nb1, ewc, wa, wb, eb0):
    rb = 1024
    grid = BN // rb
    return pl.pallas_call(
        _prep_body,
        grid=(grid,),
        in_specs=[
            pl.BlockSpec((rb, TD), lambda i: (i, 0)),
            pl.BlockSpec((rb, D), lambda i: (i, 0)),
            pl.BlockSpec((TD, DM), lambda i: (0, 0)),
            pl.BlockSpec((1, DM), lambda i: (0, 0)),
            pl.BlockSpec((DM, DM), lambda i: (0, 0)),
            pl.BlockSpec((1, DM), lambda i: (0, 0)),
            pl.BlockSpec((D, DM), lambda i: (0, 0)),
            pl.BlockSpec((DM, DM), lambda i: (0, 0)),
            pl.BlockSpec((DM, DM), lambda i: (0, 0)),
            pl.BlockSpec((1, DM), lambda i: (0, 0)),
        ],
        out_specs=[
            pl.BlockSpec((rb, DM), lambda i: (i, 0)),
            pl.BlockSpec((rb, DM), lambda i: (i, 0)),
            pl.BlockSpec((rb, DM), lambda i: (i, 0)),
            pl.BlockSpec((rb, DM), lambda i: (i, 0)),
            pl.BlockSpec((rb, VD), lambda i: (i, 0)),
        ],
        out_shape=[
            jax.ShapeDtypeStruct((BN, DM), jnp.float32),
            jax.ShapeDtypeStruct((BN, DM), jnp.float32),
            jax.ShapeDtypeStruct((BN, DM), jnp.float32),
            jax.ShapeDtypeStruct((BN, DM), jnp.bfloat16),
            jax.ShapeDtypeStruct((BN, VD), jnp.float32),
        ],
    )(pos_t, mp_bn, nw0, nb0, nw1, nb1, ewc, wa, wb, eb0)


def _knn_body(mp_ref, mpt_ref, jl_ref, jg_ref):
    b = pl.program_id(0)
    sq = []
    for d in range(D):
        r = mp_ref[0, :, d:d + 1]
        c = mpt_ref[0, d:d + 1, :]
        sq.append((r - c) ** 2)
    dist = (sq[0] + sq[1]) + sq[2]
    iota = lax.broadcasted_iota(jnp.int32, (N, N), 1)
    big_f = jnp.float32(3e38)
    big_i = jnp.int32(1 << 30)
    cur = dist
    js = []
    for _ in range(K):
        m = jnp.min(cur, axis=1, keepdims=True)
        cand = jnp.where(cur == m, iota, big_i)
        j = jnp.min(cand, axis=1, keepdims=True)
        js.append(j)
        cur = jnp.where(iota == j, big_f, cur)
    jl = jnp.concatenate(js, axis=1)
    jl_ref[0] = jl
    jg_ref[0] = jl + b * N


def _knn(mean_pos, mpt):
    return pl.pallas_call(
        _knn_body,
        grid=(B,),
        in_specs=[
            pl.BlockSpec((1, N, D), lambda i: (i, 0, 0)),
            pl.BlockSpec((1, D, N), lambda i: (i, 0, 0)),
        ],
        out_specs=[
            pl.BlockSpec((1, N, K), lambda i: (i, 0, 0)),
            pl.BlockSpec((1, N, K), lambda i: (i, 0, 0)),
        ],
        out_shape=[
            jax.ShapeDtypeStruct((B, N, K), jnp.int32),
            jax.ShapeDtypeStruct((B, N, K), jnp.int32),
        ],
    )(mean_pos, mpt)


@functools.partial(
    pl.kernel,
    mesh=_mesh,
    compiler_params=_sc_params,
    out_type=jax.ShapeDtypeStruct((E, DM // 2), jnp.int32),
    scratch_types=[
        pltpu.VMEM((_EPW,), jnp.int32),
        pltpu.VMEM((_GCH, DM // 2), jnp.int32),
        pltpu.VMEM((_GCH, DM // 2), jnp.int32),
        pltpu.SemaphoreType.DMA,
        pltpu.SemaphoreType.DMA,
    ],
)
def _gather_rows(t_hbm, idx_hbm, out_hbm, idx_v, rows0, rows1, sem0, sem1):
    wid = lax.axis_index("s") * 2 + lax.axis_index("c")
    base = wid * _EPW
    pltpu.sync_copy(idx_hbm.at[pl.ds(base, _EPW)], idx_v)
    nch = _EPW // _GCH

    def gsrc(ci):
        return t_hbm.at[idx_v.at[pl.ds(ci * _GCH, _GCH)]]

    def stage(ci, buf, sem):
        pltpu.make_async_copy(gsrc(ci), buf, sem).wait()
        pltpu.sync_copy(buf, out_hbm.at[pl.ds(base + ci * _GCH, _GCH)])

    pltpu.async_copy(gsrc(0), rows0, sem0)
    pltpu.async_copy(gsrc(1), rows1, sem1)

    def body(i, carry):
        c0 = i * 2
        stage(c0, rows0, sem0)
        pltpu.async_copy(gsrc(c0 + 2), rows0, sem0)
        stage(c0 + 1, rows1, sem1)
        pltpu.async_copy(gsrc(c0 + 3), rows1, sem1)
        return carry

    lax.fori_loop(0, nch // 2 - 1, body, 0)
    stage(nch - 2, rows0, sem0)
    stage(nch - 1, rows1, sem1)


def _edge_mlp_body(s_ref, tg_ref, w1_ref, b1_ref, w2_ref, attn_ref):
    s = s_ref[...]
    tg = tg_ref[...].astype(jnp.float32)
    z1 = jnp.maximum(tg + s[:, None, :], 0.0)
    z1f = z1.reshape(z1.shape[0] * K, DM).astype(jnp.bfloat16)
    z2 = jnp.maximum(
        jnp.dot(z1f, w1_ref[...], preferred_element_type=jnp.float32)
        + b1_ref[...], 0.0)
    z23 = z2.reshape(z1.shape[0], K, DM)
    logits = jnp.sum(z23 * w2_ref[...], axis=-1)
    m = jnp.max(logits, axis=-1, keepdims=True)
    e = jnp.exp(logits - m)
    attn_ref[...] = e / jnp.sum(e, axis=-1, keepdims=True)


def _edge_mlp(s, tg, w1, b1, w2r):
    nb = 128
    return pl.pallas_call(
        _edge_mlp_body,
        grid=(BN // nb,),
        in_specs=[
            pl.BlockSpec((nb, DM), lambda i: (i, 0)),
            pl.BlockSpec((nb, K, DM), lambda i: (i, 0, 0)),
            pl.BlockSpec((DM, DM), lambda i: (0, 0)),
            pl.BlockSpec((1, DM), lambda i: (0, 0)),
            pl.BlockSpec((1, 1, DM), lambda i: (0, 0, 0)),
        ],
        out_specs=pl.BlockSpec((nb, K), lambda i: (i, 0)),
        out_shape=jax.ShapeDtypeStruct((BN, K), jnp.float32),
    )(s, tg, w1, b1, w2r)


@functools.partial(
    pl.kernel,
    mesh=_mesh,
    compiler_params=_sc_params,
    out_type=jax.ShapeDtypeStruct((BN, N), jnp.float32),
    scratch_types=[
        pltpu.VMEM((_RPW, K), jnp.float32),
        pltpu.VMEM((_RPW, K), jnp.int32),
        pltpu.VMEM((N,), jnp.float32),
    ],
)
def _build_a(attn_hbm, jloc_hbm, a_hbm, attn_v, idx_v, row_v):
    wid = lax.axis_index("s") * 2 + lax.axis_index("c")
    base = wid * _RPW
    pltpu.sync_copy(attn_hbm.at[pl.ds(base, _RPW)], attn_v)
    pltpu.sync_copy(jloc_hbm.at[pl.ds(base, _RPW)], idx_v)
    zeros = jnp.zeros((16,), jnp.float32)

    def zbody(i, carry):
        row_v[pl.ds(i * 16, 16)] = zeros
        return carry

    lax.fori_loop(0, N // 16, zbody, 0)

    def rbody(r, carry):
        av = attn_v[r]
        iv = idx_v[r]
        plsc.store_scatter(row_v, [iv], av)
        pltpu.sync_copy(row_v, a_hbm.at[base + r])
        plsc.store_scatter(row_v, [iv], zeros)
        return carry

    lax.fori_loop(0, _RPW, rbody, 0)


def _update_body(a_ref, hb_ref, hblk_ref, c_ref, wa_ref, wb_ref, b0_ref,
                 w1_ref, b1_ref, ewa_ref, ewb_ref, eb0_ref,
                 h_ref, s_ref, t_ref):
    agg = jnp.dot(a_ref[...], hb_ref[0], preferred_element_type=jnp.float32)
    z = jnp.maximum(
        jnp.dot(hblk_ref[...], wa_ref[...], preferred_element_type=jnp.float32)
        + jnp.dot(agg, wb_ref[...], preferred_element_type=jnp.float32)
        + b0_ref[...], 0.0)
    h = (jnp.dot(z, w1_ref[...], preferred_element_type=jnp.float32)
         + b1_ref[...])
    h_ref[...] = h
    c = c_ref[...]
    s_ref[...] = (jnp.dot(h, ewa_ref[...], preferred_element_type=jnp.float32)
                  + c + eb0_ref[...])
    t_ref[...] = (jnp.dot(h, ewb_ref[...], preferred_element_type=jnp.float32)
                  - c).astype(jnp.bfloat16)


def _update(a, h3, h, c, wa, wb, b0, w1, b1, ewa, ewb, eb0):
    rb = 256
    blocks_per_b = N // rb
    wspec = pl.BlockSpec((DM, DM), lambda i: (0, 0))
    bspec = pl.BlockSpec((1, DM), lambda i: (0, 0))
    return pl.pallas_call(
        _update_body,
        grid=(BN // rb,),
        in_specs=[
            pl.BlockSpec((rb, N), lambda i: (i, 0)),
            pl.BlockSpec((1, N, DM), lambda i: (i // blocks_per_b, 0, 0)),
            pl.BlockSpec((rb, DM), lambda i: (i, 0)),
            pl.BlockSpec((rb, DM), lambda i: (i, 0)),
            wspec, wspec, bspec, wspec, bspec, wspec, wspec, bspec,
        ],
        out_specs=[
            pl.BlockSpec((rb, DM), lambda i: (i, 0)),
            pl.BlockSpec((rb, DM), lambda i: (i, 0)),
            pl.BlockSpec((rb, DM), lambda i: (i, 0)),
        ],
        out_shape=[
            jax.ShapeDtypeStruct((BN, DM), jnp.float32),
            jax.ShapeDtypeStruct((BN, DM), jnp.float32),
            jax.ShapeDtypeStruct((BN, DM), jnp.bfloat16),
        ],
    )(a, h3, h, c, wa, wb, b0, w1, b1, ewa, ewb, eb0)


def _neumann_body(a_ref, d3_ref, d_ref, out_ref, vbuf, acc):
    lvl = pl.program_id(0)
    i = pl.program_id(1)
    b = i // (N // 256)
    rows = i * 256
    brow = (b % B) * N
    vprev = vbuf[(lvl + 1) % 2, pl.ds(brow, N), :]
    vsrc = jnp.where(lvl == 0, d3_ref[0], vprev)
    vnew = jnp.dot(a_ref[...], vsrc, preferred_element_type=jnp.float32)
    vbuf[lvl % 2, pl.ds(rows, 256), :] = vnew
    prev_acc = jnp.where(lvl == 0, d_ref[...], acc[pl.ds(rows, 256), :])
    acc_new = prev_acc + vnew
    acc[pl.ds(rows, 256), :] = acc_new
    out_ref[...] = acc_new


def _neumann_all(a, delta3, delta):
    rb = 256
    blocks_per_b = N // rb
    return pl.pallas_call(
        _neumann_body,
        grid=(L, BN // rb),
        in_specs=[
            pl.BlockSpec((rb, N), lambda l, i: (i, 0)),
            pl.BlockSpec((1, N, VD), lambda l, i: (i // blocks_per_b, 0, 0)),
            pl.BlockSpec((rb, VD), lambda l, i: (i, 0)),
        ],
        out_specs=pl.BlockSpec((rb, VD), lambda l, i: (i, 0)),
        out_shape=jax.ShapeDtypeStruct((BN, VD), jnp.float32),
        scratch_shapes=[
            pltpu.VMEM((2, BN, VD), jnp.float32),
            pltpu.VMEM((BN, VD), jnp.float32),
        ],
    )(a, delta3, delta)


def kernel(positions, node_w0, node_b0, node_w1, node_b1, edge_w0, edge_b0,
           edge_w1, edge_b1, edge_w2, edge_b2,
           u0_w0, u0_b0, u0_w1, u0_b1, u1_w0, u1_b0, u1_w1, u1_b1):
    pos_t = jnp.transpose(positions, (0, 2, 1, 3)).reshape(BN, TD)
    mean_pos = positions.mean(axis=1)
    mpt = jnp.transpose(mean_pos, (0, 2, 1))
    mp_bn = mean_pos.reshape(BN, D)

    ew0a = edge_w0[:DM]
    ew0b = edge_w0[DM:2 * DM]
    ew0c = edge_w0[2 * DM:]
    eb0 = edge_b0.reshape(1, DM)
    eb1 = edge_b1.reshape(1, DM)
    w2r = edge_w2.reshape(1, 1, DM)

    h, c, s, t, delta = _prep(pos_t, mp_bn, node_w0, node_b0.reshape(1, DM),
                              node_w1, node_b1.reshape(1, DM), ew0c,
                              ew0a, ew0b, eb0)
    jloc, jglob = _knn(mean_pos, mpt)
    jloc_bn = jloc.reshape(BN, K)
    jglob_flat = jglob.reshape(E)

    updates = ((u0_w0, u0_b0, u0_w1, u0_b1), (u1_w0, u1_b0, u1_w1, u1_b1))
    ew1b = edge_w1.astype(jnp.bfloat16)
    a = None
    for r in range(3):
        t_packed = lax.bitcast_convert_type(
            t.reshape(BN, DM // 2, 2), jnp.int32)
        tg_packed = _gather_rows(t_packed, jglob_flat)
        tg = lax.bitcast_convert_type(
            tg_packed, jnp.bfloat16).reshape(BN, K, DM)
        attn = _edge_mlp(s, tg, ew1b, eb1, w2r)
        a = _build_a(attn, jloc_bn)
        if r < 2:
            w0, b0, w1, b1 = updates[r]
            h3 = h.reshape(B, N, DM)
            h, s, t = _update(a, h3, h, c, w0[:DM], w0[DM:],
                              b0.reshape(1, DM), w1, b1.reshape(1, DM),
                              ew0a, ew0b, eb0)

    delta3 = delta.reshape(B, N, VD)
    acc = _neumann_all(a, delta3, delta)
    return acc.reshape(B, N, T - 1, D).transpose(0, 2, 1, 3)


# speedup vs baseline: 5.8041x; 5.8041x over previous
import dataclasses
import functools

import jax
import jax.numpy as jnp
from jax import lax
from jax.experimental import pallas as pl
from jax.experimental.pallas import tpu as pltpu
from jax.experimental.pallas import tpu_sc as plsc

B, T, N, D = 8, 24, 1024, 3
DM = 256
K = 16
L = 4
BN = B * N
E = BN * K
TD = T * D
VD = (T - 1) * D

_NW = 32
_EPW = E // _NW
_GCH = 128
_RPW = BN // _NW

_mesh = plsc.VectorSubcoreMesh(core_axis_name="c", subcore_axis_name="s")

_sc_params = pltpu.CompilerParams()
if "needs_layout_passes" in pltpu.CompilerParams.__dataclass_fields__:
    _sc_params = dataclasses.replace(_sc_params, needs_layout_passes=False)

_HD = DM // 2


def _pack_bf16_pair(x):
    def rne_hi16(v):
        bu = lax.bitcast_convert_type(v, jnp.uint32)
        r = bu + jnp.uint32(0x7FFF) + ((bu >> 16) & jnp.uint32(1))
        return r & jnp.uint32(0xFFFF0000)

    hi = rne_hi16(x[:, :_HD])
    lo = rne_hi16(x[:, _HD:])
    return lax.bitcast_convert_type(hi | (lo >> 16), jnp.int32)


def _unpack_bf16_pair(p):
    pu = lax.bitcast_convert_type(p, jnp.uint32)
    hi = lax.bitcast_convert_type(pu & jnp.uint32(0xFFFF0000), jnp.float32)
    lo = lax.bitcast_convert_type(pu << 16, jnp.float32)
    return hi, lo


def _prep_body(pos_ref, mp_ref, nw0_ref, nb0_ref, nw1_ref, nb1_ref, ewc_ref,
               wa_ref, wb_ref, eb0_ref,
               h_ref, c_ref, s_ref, t_ref, delta_ref):
    x = pos_ref[...]
    h1 = jnp.maximum(
        jnp.dot(x, nw0_ref[...], preferred_element_type=jnp.float32)
        + nb0_ref[...], 0.0)
    h = (jnp.dot(h1, nw1_ref[...], preferred_element_type=jnp.float32)
         + nb1_ref[...])
    h_ref[...] = h
    c = jnp.dot(mp_ref[...], ewc_ref[...], preferred_element_type=jnp.float32)
    c_ref[...] = c
    s_ref[...] = (jnp.dot(h, wa_ref[...], preferred_element_type=jnp.float32)
                  + c + eb0_ref[...])
    t_ref[...] = _pack_bf16_pair(
        jnp.dot(h, wb_ref[...], preferred_element_type=jnp.float32) - c)
    delta_ref[...] = x[:, D:] - x[:, :TD - D]


def _prep(pos_t, mp_bn, nw0, nb0, nw1, nb1, ewc, wa, wb, eb0):
    rb = 1024
    grid = BN // rb
    return pl.pallas_call(
        _prep_body,
        grid=(grid,),
        in_specs=[
            pl.BlockSpec((rb, TD), lambda i: (i, 0)),
            pl.BlockSpec((rb, D), lambda i: (i, 0)),
            pl.BlockSpec((TD, DM), lambda i: (0, 0)),
            pl.BlockSpec((1, DM), lambda i: (0, 0)),
            pl.BlockSpec((DM, DM), lambda i: (0, 0)),
            pl.BlockSpec((1, DM), lambda i: (0, 0)),
            pl.BlockSpec((D, DM), lambda i: (0, 0)),
            pl.BlockSpec((DM, DM), lambda i: (0, 0)),
            pl.BlockSpec((DM, DM), lambda i: (0, 0)),
            pl.BlockSpec((1, DM), lambda i: (0, 0)),
        ],
        out_specs=[
            pl.BlockSpec((rb, DM), lambda i: (i, 0)),
            pl.BlockSpec((rb, DM), lambda i: (i, 0)),
            pl.BlockSpec((rb, DM), lambda i: (i, 0)),
            pl.BlockSpec((rb, _HD), lambda i: (i, 0)),
            pl.BlockSpec((rb, VD), lambda i: (i, 0)),
        ],
        out_shape=[
            jax.ShapeDtypeStruct((BN, DM), jnp.float32),
            jax.ShapeDtypeStruct((BN, DM), jnp.float32),
            jax.ShapeDtypeStruct((BN, DM), jnp.float32),
            jax.ShapeDtypeStruct((BN, _HD), jnp.int32),
            jax.ShapeDtypeStruct((BN, VD), jnp.float32),
        ],
    )(pos_t, mp_bn, nw0, nb0, nw1, nb1, ewc, wa, wb, eb0)


def _knn_body(mp_ref, mpt_ref, jl_ref, jg_ref):
    b = pl.program_id(0)
    sq = []
    for d in range(D):
        r = mp_ref[0, :, d:d + 1]
        c = mpt_ref[0, d:d + 1, :]
        sq.append((r - c) ** 2)
    dist = (sq[0] + sq[1]) + sq[2]
    iota = lax.broadcasted_iota(jnp.int32, (N, N), 1)
    big_f = jnp.float32(3e38)
    big_i = jnp.int32(1 << 30)
    cur = dist
    js = []
    for _ in range(K):
        m = jnp.min(cur, axis=1, keepdims=True)
        cand = jnp.where(cur == m, iota, big_i)
        j = jnp.min(cand, axis=1, keepdims=True)
        js.append(j)
        cur = jnp.where(iota == j, big_f, cur)
    jl = jnp.concatenate(js, axis=1)
    jl_ref[0] = jl
    jg_ref[0] = jl + b * N


def _knn(mean_pos, mpt):
    return pl.pallas_call(
        _knn_body,
        grid=(B,),
        in_specs=[
            pl.BlockSpec((1, N, D), lambda i: (i, 0, 0)),
            pl.BlockSpec((1, D, N), lambda i: (i, 0, 0)),
        ],
        out_specs=[
            pl.BlockSpec((1, N, K), lambda i: (i, 0, 0)),
            pl.BlockSpec((1, N, K), lambda i: (i, 0, 0)),
        ],
        out_shape=[
            jax.ShapeDtypeStruct((B, N, K), jnp.int32),
            jax.ShapeDtypeStruct((B, N, K), jnp.int32),
        ],
    )(mean_pos, mpt)


@functools.partial(
    pl.kernel,
    mesh=_mesh,
    compiler_params=_sc_params,
    out_type=jax.ShapeDtypeStruct((E, _HD), jnp.int32),
    scratch_types=[
        pltpu.VMEM((_EPW,), jnp.int32),
        pltpu.VMEM((_GCH, _HD), jnp.int32),
        pltpu.VMEM((_GCH, _HD), jnp.int32),
        pltpu.SemaphoreType.DMA,
        pltpu.SemaphoreType.DMA,
    ],
)
def _gather_rows(t_hbm, idx_hbm, out_hbm, idx_v, rows0, rows1, sem0, sem1):
    wid = lax.axis_index("s") * 2 + lax.axis_index("c")
    base = wid * _EPW
    pltpu.sync_copy(idx_hbm.at[pl.ds(base, _EPW)], idx_v)
    nch = _EPW // _GCH

    def gsrc(ci):
        return t_hbm.at[idx_v.at[pl.ds(ci * _GCH, _GCH)]]

    def stage(ci, buf, sem):
        pltpu.make_async_copy(gsrc(ci), buf, sem).wait()
        pltpu.sync_copy(buf, out_hbm.at[pl.ds(base + ci * _GCH, _GCH)])

    pltpu.async_copy(gsrc(0), rows0, sem0)
    pltpu.async_copy(gsrc(1), rows1, sem1)

    def body(i, carry):
        c0 = i * 2
        stage(c0, rows0, sem0)
        pltpu.async_copy(gsrc(c0 + 2), rows0, sem0)
        stage(c0 + 1, rows1, sem1)
        pltpu.async_copy(gsrc(c0 + 3), rows1, sem1)
        return carry

    lax.fori_loop(0, nch // 2 - 1, body, 0)
    stage(nch - 2, rows0, sem0)
    stage(nch - 1, rows1, sem1)


def _edge_mlp_body(s_ref, tg_ref, w1_ref, b1_ref, w2_ref, attn_ref):
    s = s_ref[...]
    nb = s.shape[0]
    thi, tlo = _unpack_bf16_pair(tg_ref[...])
    z1h = jnp.maximum(thi + s[:, None, :_HD], 0.0)
    z1l = jnp.maximum(tlo + s[:, None, _HD:], 0.0)
    z1hf = z1h.reshape(nb * K, _HD).astype(jnp.bfloat16)
    z1lf = z1l.reshape(nb * K, _HD).astype(jnp.bfloat16)
    z2 = jnp.maximum(
        jnp.dot(z1hf, w1_ref[:_HD, :], preferred_element_type=jnp.float32)
        + jnp.dot(z1lf, w1_ref[_HD:, :], preferred_element_type=jnp.float32)
        + b1_ref[...], 0.0)
    z23 = z2.reshape(nb, K, DM)
    logits = jnp.sum(z23 * w2_ref[...], axis=-1)
    m = jnp.max(logits, axis=-1, keepdims=True)
    e = jnp.exp(logits - m)
    attn_ref[...] = e / jnp.sum(e, axis=-1, keepdims=True)


def _edge_mlp(s, tg, w1, b1, w2r):
    nb = 128
    return pl.pallas_call(
        _edge_mlp_body,
        grid=(BN // nb,),
        in_specs=[
            pl.BlockSpec((nb, DM), lambda i: (i, 0)),
            pl.BlockSpec((nb, K, _HD), lambda i: (i, 0, 0)),
            pl.BlockSpec((DM, DM), lambda i: (0, 0)),
            pl.BlockSpec((1, DM), lambda i: (0, 0)),
            pl.BlockSpec((1, 1, DM), lambda i: (0, 0, 0)),
        ],
        out_specs=pl.BlockSpec((nb, K), lambda i: (i, 0)),
        out_shape=jax.ShapeDtypeStruct((BN, K), jnp.float32),
    )(s, tg, w1, b1, w2r)


@functools.partial(
    pl.kernel,
    mesh=_mesh,
    compiler_params=_sc_params,
    out_type=jax.ShapeDtypeStruct((BN, N), jnp.float32),
    scratch_types=[
        pltpu.VMEM((_RPW, K), jnp.float32),
        pltpu.VMEM((_RPW, K), jnp.int32),
        pltpu.VMEM((N,), jnp.float32),
    ],
)
def _build_a(attn_hbm, jloc_hbm, a_hbm, attn_v, idx_v, row_v):
    wid = lax.axis_index("s") * 2 + lax.axis_index("c")
    base = wid * _RPW
    pltpu.sync_copy(attn_hbm.at[pl.ds(base, _RPW)], attn_v)
    pltpu.sync_copy(jloc_hbm.at[pl.ds(base, _RPW)], idx_v)
    zeros = jnp.zeros((16,), jnp.float32)

    def zbody(i, carry):
        row_v[pl.ds(i * 16, 16)] = zeros
        return carry

    lax.fori_loop(0, N // 16, zbody, 0)

    def rbody(r, carry):
        av = attn_v[r]
        iv = idx_v[r]
        plsc.store_scatter(row_v, [iv], av)
        pltpu.sync_copy(row_v, a_hbm.at[base + r])
        plsc.store_scatter(row_v, [iv], zeros)
        return carry

    lax.fori_loop(0, _RPW, rbody, 0)


def _update_body(a_ref, hb_ref, hblk_ref, c_ref, wa_ref, wb_ref, b0_ref,
                 w1_ref, b1_ref, ewa_ref, ewb_ref, eb0_ref,
                 h_ref, s_ref, t_ref):
    agg = jnp.dot(a_ref[...], hb_ref[0], preferred_element_type=jnp.float32)
    z = jnp.maximum(
        jnp.dot(hblk_ref[...], wa_ref[...], preferred_element_type=jnp.float32)
        + jnp.dot(agg, wb_ref[...], preferred_element_type=jnp.float32)
        + b0_ref[...], 0.0)
    h = (jnp.dot(z, w1_ref[...], preferred_element_type=jnp.float32)
         + b1_ref[...])
    h_ref[...] = h
    c = c_ref[...]
    s_ref[...] = (jnp.dot(h, ewa_ref[...], preferred_element_type=jnp.float32)
                  + c + eb0_ref[...])
    t_ref[...] = _pack_bf16_pair(
        jnp.dot(h, ewb_ref[...], preferred_element_type=jnp.float32) - c)


def _update(a, h3, h, c, wa, wb, b0, w1, b1, ewa, ewb, eb0):
    rb = 256
    blocks_per_b = N // rb
    wspec = pl.BlockSpec((DM, DM), lambda i: (0, 0))
    bspec = pl.BlockSpec((1, DM), lambda i: (0, 0))
    return pl.pallas_call(
        _update_body,
        grid=(BN // rb,),
        in_specs=[
            pl.BlockSpec((rb, N), lambda i: (i, 0)),
            pl.BlockSpec((1, N, DM), lambda i: (i // blocks_per_b, 0, 0)),
            pl.BlockSpec((rb, DM), lambda i: (i, 0)),
            pl.BlockSpec((rb, DM), lambda i: (i, 0)),
            wspec, wspec, bspec, wspec, bspec, wspec, wspec, bspec,
        ],
        out_specs=[
            pl.BlockSpec((rb, DM), lambda i: (i, 0)),
            pl.BlockSpec((rb, DM), lambda i: (i, 0)),
            pl.BlockSpec((rb, _HD), lambda i: (i, 0)),
        ],
        out_shape=[
            jax.ShapeDtypeStruct((BN, DM), jnp.float32),
            jax.ShapeDtypeStruct((BN, DM), jnp.float32),
            jax.ShapeDtypeStruct((BN, _HD), jnp.int32),
        ],
    )(a, h3, h, c, wa, wb, b0, w1, b1, ewa, ewb, eb0)


def _neumann_body(a_ref, d3_ref, d_ref, out_ref, vbuf, acc):
    lvl = pl.program_id(0)
    i = pl.program_id(1)
    b = i // (N // 256)
    rows = i * 256
    brow = (b % B) * N
    vprev = vbuf[(lvl + 1) % 2, pl.ds(brow, N), :]
    vsrc = jnp.where(lvl == 0, d3_ref[0], vprev)
    vnew = jnp.dot(a_ref[...], vsrc, preferred_element_type=jnp.float32)
    vbuf[lvl % 2, pl.ds(rows, 256), :] = vnew
    prev_acc = jnp.where(lvl == 0, d_ref[...], acc[pl.ds(rows, 256), :])
    acc_new = prev_acc + vnew
    acc[pl.ds(rows, 256), :] = acc_new
    out_ref[...] = acc_new


def _neumann_all(a, delta3, delta):
    rb = 256
    blocks_per_b = N // rb
    return pl.pallas_call(
        _neumann_body,
        grid=(L, BN // rb),
        in_specs=[
            pl.BlockSpec((rb, N), lambda l, i: (i, 0)),
            pl.BlockSpec((1, N, VD), lambda l, i: (i // blocks_per_b, 0, 0)),
            pl.BlockSpec((rb, VD), lambda l, i: (i, 0)),
        ],
        out_specs=pl.BlockSpec((rb, VD), lambda l, i: (i, 0)),
        out_shape=jax.ShapeDtypeStruct((BN, VD), jnp.float32),
        scratch_shapes=[
            pltpu.VMEM((2, BN, VD), jnp.float32),
            pltpu.VMEM((BN, VD), jnp.float32),
        ],
    )(a, delta3, delta)


def kernel(positions, node_w0, node_b0, node_w1, node_b1, edge_w0, edge_b0,
           edge_w1, edge_b1, edge_w2, edge_b2,
           u0_w0, u0_b0, u0_w1, u0_b1, u1_w0, u1_b0, u1_w1, u1_b1):
    pos_t = jnp.transpose(positions, (0, 2, 1, 3)).reshape(BN, TD)
    mean_pos = positions.mean(axis=1)
    mpt = jnp.transpose(mean_pos, (0, 2, 1))
    mp_bn = mean_pos.reshape(BN, D)

    ew0a = edge_w0[:DM]
    ew0b = edge_w0[DM:2 * DM]
    ew0c = edge_w0[2 * DM:]
    eb0 = edge_b0.reshape(1, DM)
    eb1 = edge_b1.reshape(1, DM)
    w2r = edge_w2.reshape(1, 1, DM)

    h, c, s, t, delta = _prep(pos_t, mp_bn, node_w0, node_b0.reshape(1, DM),
                              node_w1, node_b1.reshape(1, DM), ew0c,
                              ew0a, ew0b, eb0)
    jloc, jglob = _knn(mean_pos, mpt)
    jloc_bn = jloc.reshape(BN, K)
    jglob_flat = jglob.reshape(E)

    updates = ((u0_w0, u0_b0, u0_w1, u0_b1), (u1_w0, u1_b0, u1_w1, u1_b1))
    ew1b = edge_w1.astype(jnp.bfloat16)
    a = None
    for r in range(3):
        tg = _gather_rows(t, jglob_flat).reshape(BN, K, _HD)
        attn = _edge_mlp(s, tg, ew1b, eb1, w2r)
        a = _build_a(attn, jloc_bn)
        if r < 2:
            w0, b0, w1, b1 = updates[r]
            h3 = h.reshape(B, N, DM)
            h, s, t = _update(a, h3, h, c, w0[:DM], w0[DM:],
                              b0.reshape(1, DM), w1, b1.reshape(1, DM),
                              ew0a, ew0b, eb0)

    delta3 = delta.reshape(B, N, VD)
    acc = _neumann_all(a, delta3, delta)
    return acc.reshape(B, N, T - 1, D).transpose(0, 2, 1, 3)


# speedup vs baseline: 6.0090x; 1.0353x over previous
import dataclasses
import functools

import jax
import jax.numpy as jnp
from jax import lax
from jax.experimental import pallas as pl
from jax.experimental.pallas import tpu as pltpu
from jax.experimental.pallas import tpu_sc as plsc

B, T, N, D = 8, 24, 1024, 3
DM = 256
K = 16
L = 4
BN = B * N
E = BN * K
TD = T * D
VD = (T - 1) * D

_NW = 32
_EPW = E // _NW
_GCH = 128
_RPW = BN // _NW

_mesh = plsc.VectorSubcoreMesh(core_axis_name="c", subcore_axis_name="s")

_sc_params = pltpu.CompilerParams()
if "needs_layout_passes" in pltpu.CompilerParams.__dataclass_fields__:
    _sc_params = dataclasses.replace(_sc_params, needs_layout_passes=False)

_HD = DM // 2


def _pack_bf16_pair(x):
    def rne_hi16(v):
        bu = lax.bitcast_convert_type(v, jnp.uint32)
        r = bu + jnp.uint32(0x7FFF) + ((bu >> 16) & jnp.uint32(1))
        return r & jnp.uint32(0xFFFF0000)

    hi = rne_hi16(x[:, :_HD])
    lo = rne_hi16(x[:, _HD:])
    return lax.bitcast_convert_type(hi | (lo >> 16), jnp.int32)


def _unpack_bf16_pair(p):
    pu = lax.bitcast_convert_type(p, jnp.uint32)
    hi = lax.bitcast_convert_type(pu & jnp.uint32(0xFFFF0000), jnp.float32)
    lo = lax.bitcast_convert_type(pu << 16, jnp.float32)
    return hi, lo


def _prep_body(pos_ref, mp_ref, nw0_ref, nb0_ref, nw1_ref, nb1_ref, ewc_ref,
               wa_ref, wb_ref, eb0_ref,
               h_ref, c_ref, s_ref, t_ref, delta_ref):
    x = pos_ref[...]
    h1 = jnp.maximum(
        jnp.dot(x, nw0_ref[...], preferred_element_type=jnp.float32)
        + nb0_ref[...], 0.0)
    h = (jnp.dot(h1, nw1_ref[...], preferred_element_type=jnp.float32)
         + nb1_ref[...])
    h_ref[...] = h
    c = jnp.dot(mp_ref[...], ewc_ref[...], preferred_element_type=jnp.float32)
    c_ref[...] = c
    s_ref[...] = (jnp.dot(h, wa_ref[...], preferred_element_type=jnp.float32)
                  + c + eb0_ref[...])
    t_ref[...] = _pack_bf16_pair(
        jnp.dot(h, wb_ref[...], preferred_element_type=jnp.float32) - c)
    delta_ref[...] = x[:, D:] - x[:, :TD - D]


def _prep(pos_t, mp_bn, nw0, nb0, nw1, nb1, ewc, wa, wb, eb0):
    rb = 1024
    grid = BN // rb
    return pl.pallas_call(
        _prep_body,
        grid=(grid,),
        in_specs=[
            pl.BlockSpec((rb, TD), lambda i: (i, 0)),
            pl.BlockSpec((rb, D), lambda i: (i, 0)),
            pl.BlockSpec((TD, DM), lambda i: (0, 0)),
            pl.BlockSpec((1, DM), lambda i: (0, 0)),
            pl.BlockSpec((DM, DM), lambda i: (0, 0)),
            pl.BlockSpec((1, DM), lambda i: (0, 0)),
            pl.BlockSpec((D, DM), lambda i: (0, 0)),
            pl.BlockSpec((DM, DM), lambda i: (0, 0)),
            pl.BlockSpec((DM, DM), lambda i: (0, 0)),
            pl.BlockSpec((1, DM), lambda i: (0, 0)),
        ],
        out_specs=[
            pl.BlockSpec((rb, DM), lambda i: (i, 0)),
            pl.BlockSpec((rb, DM), lambda i: (i, 0)),
            pl.BlockSpec((rb, DM), lambda i: (i, 0)),
            pl.BlockSpec((rb, _HD), lambda i: (i, 0)),
            pl.BlockSpec((rb, VD), lambda i: (i, 0)),
        ],
        out_shape=[
            jax.ShapeDtypeStruct((BN, DM), jnp.float32),
            jax.ShapeDtypeStruct((BN, DM), jnp.float32),
            jax.ShapeDtypeStruct((BN, DM), jnp.float32),
            jax.ShapeDtypeStruct((BN, _HD), jnp.int32),
            jax.ShapeDtypeStruct((BN, VD), jnp.float32),
        ],
    )(pos_t, mp_bn, nw0, nb0, nw1, nb1, ewc, wa, wb, eb0)


def _knn_body(mp_ref, mpt_ref, jl_ref, jg_ref):
    b = pl.program_id(0)
    sq = []
    for d in range(D):
        r = mp_ref[0, :, d:d + 1]
        c = mpt_ref[0, d:d + 1, :]
        sq.append((r - c) ** 2)
    dist = (sq[0] + sq[1]) + sq[2]
    iota = lax.broadcasted_iota(jnp.int32, (N, N), 1)
    big_f = jnp.float32(3e38)
    big_i = jnp.int32(1 << 30)
    cur = dist
    js = []
    for _ in range(K):
        m = jnp.min(cur, axis=1, keepdims=True)
        cand = jnp.where(cur == m, iota, big_i)
        j = jnp.min(cand, axis=1, keepdims=True)
        js.append(j)
        cur = jnp.where(iota == j, big_f, cur)
    jl = jnp.concatenate(js, axis=1)
    jl_ref[0] = jl
    jg_ref[0] = jl + b * N


def _knn(mean_pos, mpt):
    return pl.pallas_call(
        _knn_body,
        grid=(B,),
        in_specs=[
            pl.BlockSpec((1, N, D), lambda i: (i, 0, 0)),
            pl.BlockSpec((1, D, N), lambda i: (i, 0, 0)),
        ],
        out_specs=[
            pl.BlockSpec((1, N, K), lambda i: (i, 0, 0)),
            pl.BlockSpec((1, N, K), lambda i: (i, 0, 0)),
        ],
        out_shape=[
            jax.ShapeDtypeStruct((B, N, K), jnp.int32),
            jax.ShapeDtypeStruct((B, N, K), jnp.int32),
        ],
    )(mean_pos, mpt)


@functools.partial(
    pl.kernel,
    mesh=_mesh,
    compiler_params=_sc_params,
    out_type=jax.ShapeDtypeStruct((E, _HD), jnp.int32),
    scratch_types=[
        pltpu.VMEM((_EPW,), jnp.int32),
        pltpu.VMEM((_GCH, _HD), jnp.int32),
        pltpu.VMEM((_GCH, _HD), jnp.int32),
        pltpu.SemaphoreType.DMA,
        pltpu.SemaphoreType.DMA,
    ],
)
def _gather_rows(t_hbm, idx_hbm, out_hbm, idx_v, rows0, rows1, sem0, sem1):
    wid = lax.axis_index("s") * 2 + lax.axis_index("c")
    base = wid * _EPW
    pltpu.sync_copy(idx_hbm.at[pl.ds(base, _EPW)], idx_v)
    nch = _EPW // _GCH

    def gsrc(ci):
        return t_hbm.at[idx_v.at[pl.ds(ci * _GCH, _GCH)]]

    def stage(ci, buf, sem):
        pltpu.make_async_copy(gsrc(ci), buf, sem).wait()
        pltpu.sync_copy(buf, out_hbm.at[pl.ds(base + ci * _GCH, _GCH)])

    pltpu.async_copy(gsrc(0), rows0, sem0)
    pltpu.async_copy(gsrc(1), rows1, sem1)

    def body(i, carry):
        c0 = i * 2
        stage(c0, rows0, sem0)
        pltpu.async_copy(gsrc(c0 + 2), rows0, sem0)
        stage(c0 + 1, rows1, sem1)
        pltpu.async_copy(gsrc(c0 + 3), rows1, sem1)
        return carry

    lax.fori_loop(0, nch // 2 - 1, body, 0)
    stage(nch - 2, rows0, sem0)
    stage(nch - 1, rows1, sem1)


def _edge_mlp_body(s_ref, tg_ref, w1_ref, b1_ref, w2_ref, attn_ref):
    s = s_ref[...]
    nb = s.shape[0]
    thi, tlo = _unpack_bf16_pair(tg_ref[...])
    z1h = jnp.maximum(thi + s[:, None, :_HD], 0.0)
    z1l = jnp.maximum(tlo + s[:, None, _HD:], 0.0)
    z1hf = z1h.reshape(nb * K, _HD).astype(jnp.bfloat16)
    z1lf = z1l.reshape(nb * K, _HD).astype(jnp.bfloat16)
    z2 = jnp.maximum(
        jnp.dot(z1hf, w1_ref[:_HD, :], preferred_element_type=jnp.float32)
        + jnp.dot(z1lf, w1_ref[_HD:, :], preferred_element_type=jnp.float32)
        + b1_ref[...], 0.0)
    z23 = z2.reshape(nb, K, DM)
    logits = jnp.sum(z23 * w2_ref[...], axis=-1)
    m = jnp.max(logits, axis=-1, keepdims=True)
    e = jnp.exp(logits - m)
    attn_ref[...] = e / jnp.sum(e, axis=-1, keepdims=True)


def _edge_mlp(s, tg, w1, b1, w2r):
    nb = 128
    return pl.pallas_call(
        _edge_mlp_body,
        grid=(BN // nb,),
        in_specs=[
            pl.BlockSpec((nb, DM), lambda i: (i, 0)),
            pl.BlockSpec((nb, K, _HD), lambda i: (i, 0, 0)),
            pl.BlockSpec((DM, DM), lambda i: (0, 0)),
            pl.BlockSpec((1, DM), lambda i: (0, 0)),
            pl.BlockSpec((1, 1, DM), lambda i: (0, 0, 0)),
        ],
        out_specs=pl.BlockSpec((nb, K), lambda i: (i, 0)),
        out_shape=jax.ShapeDtypeStruct((BN, K), jnp.float32),
    )(s, tg, w1, b1, w2r)


@functools.partial(
    pl.kernel,
    mesh=_mesh,
    compiler_params=_sc_params,
    out_type=jax.ShapeDtypeStruct((BN, N), jnp.float32),
    scratch_types=[
        pltpu.VMEM((_RPW, K), jnp.float32),
        pltpu.VMEM((_RPW, K), jnp.int32),
        pltpu.VMEM((N,), jnp.float32),
        pltpu.VMEM((N,), jnp.float32),
        pltpu.SemaphoreType.DMA,
        pltpu.SemaphoreType.DMA,
    ],
)
def _build_a(attn_hbm, jloc_hbm, a_hbm, attn_v, idx_v, row0, row1, sem0,
             sem1):
    wid = lax.axis_index("s") * 2 + lax.axis_index("c")
    base = wid * _RPW
    pltpu.sync_copy(attn_hbm.at[pl.ds(base, _RPW)], attn_v)
    pltpu.sync_copy(jloc_hbm.at[pl.ds(base, _RPW)], idx_v)
    zeros = jnp.zeros((16,), jnp.float32)

    def zbody(i, carry):
        row0[pl.ds(i * 16, 16)] = zeros
        row1[pl.ds(i * 16, 16)] = zeros
        return carry

    lax.fori_loop(0, N // 16, zbody, 0)

    def put(r, buf, sem):
        plsc.store_scatter(buf, [idx_v[r]], attn_v[r])
        pltpu.async_copy(buf, a_hbm.at[base + r], sem)

    def clr(r, buf, sem):
        pltpu.make_async_copy(buf, a_hbm.at[base + r], sem).wait()
        plsc.store_scatter(buf, [idx_v[r]], zeros)

    put(0, row0, sem0)
    put(1, row1, sem1)

    def rbody(i, carry):
        r = i * 2
        clr(r - 2, row0, sem0)
        put(r, row0, sem0)
        clr(r - 1, row1, sem1)
        put(r + 1, row1, sem1)
        return carry

    lax.fori_loop(1, _RPW // 2, rbody, 0)
    pltpu.make_async_copy(row0, a_hbm.at[base + _RPW - 2], sem0).wait()
    pltpu.make_async_copy(row1, a_hbm.at[base + _RPW - 1], sem1).wait()


def _update_body(a_ref, hb_ref, hblk_ref, c_ref, wa_ref, wb_ref, b0_ref,
                 w1_ref, b1_ref, ewa_ref, ewb_ref, eb0_ref,
                 h_ref, s_ref, t_ref):
    agg = jnp.dot(a_ref[...], hb_ref[0], preferred_element_type=jnp.float32)
    z = jnp.maximum(
        jnp.dot(hblk_ref[...], wa_ref[...], preferred_element_type=jnp.float32)
        + jnp.dot(agg, wb_ref[...], preferred_element_type=jnp.float32)
        + b0_ref[...], 0.0)
    h = (jnp.dot(z, w1_ref[...], preferred_element_type=jnp.float32)
         + b1_ref[...])
    h_ref[...] = h
    c = c_ref[...]
    s_ref[...] = (jnp.dot(h, ewa_ref[...], preferred_element_type=jnp.float32)
                  + c + eb0_ref[...])
    t_ref[...] = _pack_bf16_pair(
        jnp.dot(h, ewb_ref[...], preferred_element_type=jnp.float32) - c)


def _update(a, h3, h, c, wa, wb, b0, w1, b1, ewa, ewb, eb0):
    rb = 256
    blocks_per_b = N // rb
    wspec = pl.BlockSpec((DM, DM), lambda i: (0, 0))
    bspec = pl.BlockSpec((1, DM), lambda i: (0, 0))
    return pl.pallas_call(
        _update_body,
        grid=(BN // rb,),
        in_specs=[
            pl.BlockSpec((rb, N), lambda i: (i, 0)),
            pl.BlockSpec((1, N, DM), lambda i: (i // blocks_per_b, 0, 0)),
            pl.BlockSpec((rb, DM), lambda i: (i, 0)),
            pl.BlockSpec((rb, DM), lambda i: (i, 0)),
            wspec, wspec, bspec, wspec, bspec, wspec, wspec, bspec,
        ],
        out_specs=[
            pl.BlockSpec((rb, DM), lambda i: (i, 0)),
            pl.BlockSpec((rb, DM), lambda i: (i, 0)),
            pl.BlockSpec((rb, _HD), lambda i: (i, 0)),
        ],
        out_shape=[
            jax.ShapeDtypeStruct((BN, DM), jnp.float32),
            jax.ShapeDtypeStruct((BN, DM), jnp.float32),
            jax.ShapeDtypeStruct((BN, _HD), jnp.int32),
        ],
    )(a, h3, h, c, wa, wb, b0, w1, b1, ewa, ewb, eb0)


def _neumann_body(a_ref, d3_ref, d_ref, out_ref, vbuf, acc):
    lvl = pl.program_id(0)
    i = pl.program_id(1)
    b = i // (N // 256)
    rows = i * 256
    brow = (b % B) * N
    vprev = vbuf[(lvl + 1) % 2, pl.ds(brow, N), :]
    vsrc = jnp.where(lvl == 0, d3_ref[0], vprev)
    vnew = jnp.dot(a_ref[...], vsrc, preferred_element_type=jnp.float32)
    vbuf[lvl % 2, pl.ds(rows, 256), :] = vnew
    prev_acc = jnp.where(lvl == 0, d_ref[...], acc[pl.ds(rows, 256), :])
    acc_new = prev_acc + vnew
    acc[pl.ds(rows, 256), :] = acc_new
    out_ref[...] = acc_new


def _neumann_all(a, delta3, delta):
    rb = 256
    blocks_per_b = N // rb
    return pl.pallas_call(
        _neumann_body,
        grid=(L, BN // rb),
        in_specs=[
            pl.BlockSpec((rb, N), lambda l, i: (i, 0)),
            pl.BlockSpec((1, N, VD), lambda l, i: (i // blocks_per_b, 0, 0)),
            pl.BlockSpec((rb, VD), lambda l, i: (i, 0)),
        ],
        out_specs=pl.BlockSpec((rb, VD), lambda l, i: (i, 0)),
        out_shape=jax.ShapeDtypeStruct((BN, VD), jnp.float32),
        scratch_shapes=[
            pltpu.VMEM((2, BN, VD), jnp.float32),
            pltpu.VMEM((BN, VD), jnp.float32),
        ],
    )(a, delta3, delta)


def kernel(positions, node_w0, node_b0, node_w1, node_b1, edge_w0, edge_b0,
           edge_w1, edge_b1, edge_w2, edge_b2,
           u0_w0, u0_b0, u0_w1, u0_b1, u1_w0, u1_b0, u1_w1, u1_b1):
    pos_t = jnp.transpose(positions, (0, 2, 1, 3)).reshape(BN, TD)
    mean_pos = positions.mean(axis=1)
    mpt = jnp.transpose(mean_pos, (0, 2, 1))
    mp_bn = mean_pos.reshape(BN, D)

    ew0a = edge_w0[:DM]
    ew0b = edge_w0[DM:2 * DM]
    ew0c = edge_w0[2 * DM:]
    eb0 = edge_b0.reshape(1, DM)
    eb1 = edge_b1.reshape(1, DM)
    w2r = edge_w2.reshape(1, 1, DM)

    h, c, s, t, delta = _prep(pos_t, mp_bn, node_w0, node_b0.reshape(1, DM),
                              node_w1, node_b1.reshape(1, DM), ew0c,
                              ew0a, ew0b, eb0)
    jloc, jglob = _knn(mean_pos, mpt)
    jloc_bn = jloc.reshape(BN, K)
    jglob_flat = jglob.reshape(E)

    updates = ((u0_w0, u0_b0, u0_w1, u0_b1), (u1_w0, u1_b0, u1_w1, u1_b1))
    ew1b = edge_w1.astype(jnp.bfloat16)
    a = None
    for r in range(3):
        tg = _gather_rows(t, jglob_flat).reshape(BN, K, _HD)
        attn = _edge_mlp(s, tg, ew1b, eb1, w2r)
        a = _build_a(attn, jloc_bn)
        if r < 2:
            w0, b0, w1, b1 = updates[r]
            h3 = h.reshape(B, N, DM)
            h, s, t = _update(a, h3, h, c, w0[:DM], w0[DM:],
                              b0.reshape(1, DM), w1, b1.reshape(1, DM),
                              ew0a, ew0b, eb0)

    delta3 = delta.reshape(B, N, VD)
    acc = _neumann_all(a, delta3, delta)
    return acc.reshape(B, N, T - 1, D).transpose(0, 2, 1, 3)
